# baseline (speedup 1.0000x reference)
import functools

import jax
import jax.numpy as jnp
from jax import lax
from jax.experimental import pallas as pl
from jax.experimental.pallas import tpu as pltpu
from jax.experimental.pallas import tpu_sc as plsc

_N = 10000
_E = 320000
_D = 128
_H = 128
_B = 64

_NP = 10240

_NC = 2
_NS = 16
_NW = _NC * _NS
_EW = _E // _NW
_CH = 80
_NCHUNK = _EW // _CH
_WC = 80
_NWC = _N // _WC
_NWK = -(-_NWC // _NS)

def _sc_agg_body(node_hbm, src_hbm, dst_hbm, out_hbm, sidx_v, didx_v, rows_v, zero_v, acc_sh, isem, gsem):
    cid = lax.axis_index("c")
    sid = lax.axis_index("s")

    def zrow(r, carry):
        for j in range(_D // 16):
            zero_v[r, pl.ds(j * 16, 16)] = jnp.zeros((16,), jnp.float32)
        return carry

    lax.fori_loop(0, _WC, zrow, 0)

    def zbody(k, carry):
        q = k * _NS + sid

        @pl.when(q < _NWC)
        def _zc():
            pltpu.sync_copy(zero_v, acc_sh.at[pl.ds(q * _WC, _WC)])

        return carry

    lax.fori_loop(0, _NWK, zbody, 0)
    plsc.subcore_barrier()

    wid = sid * _NC + cid
    ebase = wid * _EW

    def load_idx_async(j, slot):
        off = ebase + j * _CH
        pltpu.async_copy(src_hbm.at[pl.ds(off, _CH)], sidx_v.at[slot], isem.at[slot])
        pltpu.async_copy(dst_hbm.at[pl.ds(off, _CH)], didx_v.at[slot], isem.at[slot])

    def wait_idx(j, slot):
        off = ebase + j * _CH
        pltpu.make_async_copy(src_hbm.at[pl.ds(off, _CH)], sidx_v.at[slot], isem.at[slot]).wait()
        pltpu.make_async_copy(dst_hbm.at[pl.ds(off, _CH)], didx_v.at[slot], isem.at[slot]).wait()

    def fire_gather(slot):
        pltpu.async_copy(node_hbm.at[sidx_v.at[slot]], rows_v.at[slot], gsem.at[slot])

    def drain_gather(slot):
        pltpu.make_async_copy(node_hbm.at[sidx_v.at[slot]], rows_v.at[slot], gsem.at[slot]).wait()

    load_idx_async(0, 0)
    wait_idx(0, 0)
    fire_gather(0)
    load_idx_async(1, 1)

    def ebody(j, carry):
        def iteration(p, pn):
            drain_gather(p)

            @pl.when(j + 1 < _NCHUNK)
            def _next_gather():
                wait_idx(j + 1, pn)
                fire_gather(pn)

            pltpu.sync_copy(rows_v.at[p], acc_sh.at[didx_v.at[p]], add=True)

            @pl.when(j + 2 < _NCHUNK)
            def _prefetch_idx():
                load_idx_async(j + 2, p)

        @pl.when(lax.rem(j, 2) == 0)
        def _even():
            iteration(0, 1)

        @pl.when(lax.rem(j, 2) == 1)
        def _odd():
            iteration(1, 0)

        return carry

    lax.fori_loop(0, _NCHUNK, ebody, 0)
    plsc.subcore_barrier()

    def obody(k, carry):
        q = k * _NS + sid

        @pl.when(q < _NWC)
        def _oc():
            pltpu.sync_copy(
                acc_sh.at[pl.ds(q * _WC, _WC)],
                out_hbm.at[cid].at[pl.ds(q * _WC, _WC)],
            )

        return carry

    lax.fori_loop(0, _NWK, obody, 0)


def _sc_agg(node, src, dst):
    mesh = plsc.VectorSubcoreMesh(
        core_axis_name="c", subcore_axis_name="s",
        num_cores=_NC, num_subcores=_NS,
    )
    run = pl.kernel(
        _sc_agg_body,
        out_type=jax.ShapeDtypeStruct((_NC, _N, _D), jnp.float32),
        mesh=mesh,
        scratch_types=[
            pltpu.VMEM((2, _CH), jnp.int32),
            pltpu.VMEM((2, _CH), jnp.int32),
            pltpu.VMEM((2, _CH, _D), jnp.float32),
            pltpu.VMEM((_WC, _D), jnp.float32),
            pltpu.VMEM_SHARED((_N, _D), jnp.float32),
            pltpu.SemaphoreType.DMA((2,)),
            pltpu.SemaphoreType.DMA((2,)),
        ],
    )
    return run(node, src, dst)


_RT = 400
_NT = _N // _RT


def _pre_body(node_r, w1_r, b_r, nw1_r):
    nw1_r[...] = jnp.dot(node_r[...], w1_r[...], preferred_element_type=jnp.float32) + b_r[...]


def _pre(node, w1, b2d):
    return pl.pallas_call(
        _pre_body,
        grid=(_NT,),
        in_specs=[
            pl.BlockSpec((_RT, _D), lambda i: (i, 0)),
            pl.BlockSpec((_D, _D), lambda i: (0, 0)),
            pl.BlockSpec((1, _D), lambda i: (0, 0)),
        ],
        out_specs=pl.BlockSpec((_RT, _D), lambda i: (i, 0)),
        out_shape=jax.ShapeDtypeStruct((_N, _D), jnp.float32),
    )(node, w1, b2d)


def _gnn_body(nw1_r, p0_r, p1_r, w2_r, x_r):
    agg = p0_r[...] + p1_r[...]
    acc = nw1_r[...] + jnp.dot(agg, w2_r[...], preferred_element_type=jnp.float32)
    x_r[...] = jnp.maximum(acc, 0.0)


def _gnn(nw1, p0, p1, w2):
    return pl.pallas_call(
        _gnn_body,
        grid=(_NT,),
        in_specs=[
            pl.BlockSpec((_RT, _D), lambda i: (i, 0)),
            pl.BlockSpec((_RT, _D), lambda i: (i, 0)),
            pl.BlockSpec((_RT, _D), lambda i: (i, 0)),
            pl.BlockSpec((_D, _D), lambda i: (0, 0)),
        ],
        out_specs=pl.BlockSpec((_RT, _D), lambda i: (i, 0)),
        out_shape=jax.ShapeDtypeStruct((_N, _D), jnp.float32),
    )(nw1, p0, p1, w2)


def _lstm_attn_body(
    starts_s, counts_s, tmax_s,
    x_r, wi_r, wh_r, bl_r, wa_r, va_r,
    out_r, aw_r,
    h_r, c_r, xt_r, xu_r, nl_r, sc_r,
):
    h_r[...] = jnp.zeros((_B, _H), jnp.float32)
    c_r[...] = jnp.zeros((_B, _H), jnp.float32)

    def gather_rows(t, buf):
        for b in range(_B):
            idx = jnp.minimum(starts_s[b] + t, _N - 1)
            buf[pl.ds(b, 1), :] = x_r[pl.ds(idx, 1), :]

    def halfstep(t, cur, nxt):
        gates = (
            jnp.dot(cur[...], wi_r[...], preferred_element_type=jnp.float32)
            + jnp.dot(h_r[...], wh_r[...], preferred_element_type=jnp.float32)
            + bl_r[...]
        )
        gather_rows(t + 1, nxt)
        i_g = jax.nn.sigmoid(gates[:, 0:_H])
        f_g = jax.nn.sigmoid(gates[:, _H:2 * _H])
        g_g = jnp.tanh(gates[:, 2 * _H:3 * _H])
        o_g = jax.nn.sigmoid(gates[:, 3 * _H:4 * _H])
        c_new = f_g * c_r[...] + i_g * g_g
        h_new = o_g * jnp.tanh(c_new)
        h_r[...] = h_new
        c_r[...] = c_new
        for b in range(_B):
            dst_row = jnp.where(t < counts_s[b], starts_s[b] + t, _NP - 1)
            nl_r[pl.ds(dst_row, 1), :] = h_new[b:b + 1, :]

    gather_rows(0, xt_r)

    def step(t, carry):
        @pl.when(lax.rem(t, 2) == 0)
        def _even():
            halfstep(t, xt_r, xu_r)

        @pl.when(lax.rem(t, 2) == 1)
        def _odd():
            halfstep(t, xu_r, xt_r)

        return carry

    lax.fori_loop(0, tmax_s[0], step, 0)

    def score_tile(j, carry):
        rows = pl.ds(j * _RT, _RT)
        tt = jnp.tanh(jnp.dot(nl_r[rows, :], wa_r[...], preferred_element_type=jnp.float32))
        sc_r[rows, :] = jnp.sum(tt * va_r[...], axis=1, keepdims=True)
        return carry

    lax.fori_loop(0, _NT, score_tile, 0)

    iota = lax.broadcasted_iota(jnp.int32, (128, 1), 0)
    neg_inf11 = jnp.full((1, 1), -jnp.inf, jnp.float32)
    neg_inf1h = jnp.full((1, _H), -jnp.inf, jnp.float32)

    def graph_body(b, carry):
        s0 = starts_s[b]
        cnt = counts_s[b]

        @pl.when(cnt > 0)
        def _nonempty():
            ktiles = (cnt + 127) // 128

            def max_tile(k, m):
                tile = sc_r[pl.ds(s0 + k * 128, 128), :]
                msk = iota < (cnt - k * 128)
                return jnp.maximum(m, jnp.max(jnp.where(msk, tile, -jnp.inf), axis=0, keepdims=True))

            m = lax.fori_loop(0, ktiles, max_tile, neg_inf11)

            def sum_tile(k, acc):
                tile = sc_r[pl.ds(s0 + k * 128, 128), :]
                msk = iota < (cnt - k * 128)
                return acc + jnp.sum(jnp.where(msk, jnp.exp(tile - m), 0.0), axis=0, keepdims=True)

            den = lax.fori_loop(0, ktiles, sum_tile, jnp.zeros((1, 1), jnp.float32))
            inv = 1.0 / den

            def pool_tile(k, acc):
                rows = pl.ds(s0 + k * 128, 128)
                w = jnp.exp(sc_r[rows, :] - m) * inv
                aw_r[rows, :] = w
                na = nl_r[rows, :] * w
                msk = iota < (cnt - k * 128)
                na = jnp.where(msk, na, -jnp.inf)
                return jnp.maximum(acc, jnp.max(na, axis=0, keepdims=True))

            pooled = lax.fori_loop(0, ktiles, pool_tile, neg_inf1h)
            out_r[pl.ds(b, 1), :] = pooled

        @pl.when(cnt == 0)
        def _empty():
            out_r[pl.ds(b, 1), :] = jnp.zeros((1, _H), jnp.float32)

        return carry

    lax.fori_loop(0, _B, graph_body, 0)


def _lstm_attn(starts, counts, tmax, x, wi, wh, bl2d, wa, va2d):
    return pl.pallas_call(
        _lstm_attn_body,
        in_specs=[
            pl.BlockSpec(memory_space=pltpu.SMEM),
            pl.BlockSpec(memory_space=pltpu.SMEM),
            pl.BlockSpec(memory_space=pltpu.SMEM),
            pl.BlockSpec(memory_space=pltpu.VMEM),
            pl.BlockSpec(memory_space=pltpu.VMEM),
            pl.BlockSpec(memory_space=pltpu.VMEM),
            pl.BlockSpec(memory_space=pltpu.VMEM),
            pl.BlockSpec(memory_space=pltpu.VMEM),
            pl.BlockSpec(memory_space=pltpu.VMEM),
        ],
        out_specs=[
            pl.BlockSpec(memory_space=pltpu.VMEM),
            pl.BlockSpec(memory_space=pltpu.VMEM),
        ],
        out_shape=[
            jax.ShapeDtypeStruct((_B, _H), jnp.float32),
            jax.ShapeDtypeStruct((_NP, 1), jnp.float32),
        ],
        scratch_shapes=[
            pltpu.VMEM((_B, _H), jnp.float32),
            pltpu.VMEM((_B, _H), jnp.float32),
            pltpu.VMEM((_B, _D), jnp.float32),
            pltpu.VMEM((_B, _D), jnp.float32),
            pltpu.VMEM((_NP, _H), jnp.float32),
            pltpu.VMEM((_NP, 1), jnp.float32),
        ],
    )(starts, counts, tmax, x, wi, wh, bl2d, wa, va2d)


def kernel(node, edge_index, batch, W1, W2, b_gnn, Wi, Wh, b_lstm, Wa, va):
    src = edge_index[0]
    dst = edge_index[1]

    ids = jnp.arange(_B, dtype=batch.dtype)
    starts = jnp.searchsorted(batch, ids, side="left").astype(jnp.int32)
    ends = jnp.searchsorted(batch, ids, side="right").astype(jnp.int32)
    counts = ends - starts
    tmax = jnp.max(counts).reshape(1)

    parts = _sc_agg(node, src, dst)
    nw1 = _pre(node, W1, b_gnn.reshape(1, _D))
    x = _gnn(nw1, parts[0], parts[1], W2)
    out, aw = _lstm_attn(
        starts, counts, tmax, x,
        Wi, Wh, b_lstm.reshape(1, 4 * _H),
        Wa, va.reshape(1, _H),
    )
    return out, aw[:_N, 0]


# speedup vs baseline: 1.0027x; 1.0027x over previous
import functools

import jax
import jax.numpy as jnp
from jax import lax
from jax.experimental import pallas as pl
from jax.experimental.pallas import tpu as pltpu
from jax.experimental.pallas import tpu_sc as plsc

_N = 10000
_E = 320000
_D = 128
_H = 128
_B = 64

_NP = 10240

_NC = 2
_NS = 16
_NW = _NC * _NS
_EW = _E // _NW
_CH = 80
_NCHUNK = _EW // _CH
_WC = 80
_NWC = _N // _WC
_NWK = -(-_NWC // _NS)

def _sc_agg_body(node_hbm, src_hbm, dst_hbm, out_hbm, sidx_v, didx_v, rows_v, zero_v, acc_sh, isem, gsem):
    cid = lax.axis_index("c")
    sid = lax.axis_index("s")

    def zrow(r, carry):
        for j in range(_D // 16):
            zero_v[r, pl.ds(j * 16, 16)] = jnp.zeros((16,), jnp.float32)
        return carry

    lax.fori_loop(0, _WC, zrow, 0)

    def zbody(k, carry):
        q = k * _NS + sid

        @pl.when(q < _NWC)
        def _zc():
            pltpu.sync_copy(zero_v, acc_sh.at[pl.ds(q * _WC, _WC)])

        return carry

    lax.fori_loop(0, _NWK, zbody, 0)
    plsc.subcore_barrier()

    wid = sid * _NC + cid
    ebase = wid * _EW

    def load_idx_async(j, slot):
        off = ebase + j * _CH
        pltpu.async_copy(src_hbm.at[pl.ds(off, _CH)], sidx_v.at[slot], isem.at[slot])
        pltpu.async_copy(dst_hbm.at[pl.ds(off, _CH)], didx_v.at[slot], isem.at[slot])

    def wait_idx(j, slot):
        off = ebase + j * _CH
        pltpu.make_async_copy(src_hbm.at[pl.ds(off, _CH)], sidx_v.at[slot], isem.at[slot]).wait()
        pltpu.make_async_copy(dst_hbm.at[pl.ds(off, _CH)], didx_v.at[slot], isem.at[slot]).wait()

    def fire_gather(slot):
        pltpu.async_copy(node_hbm.at[sidx_v.at[slot]], rows_v.at[slot], gsem.at[slot])

    def drain_gather(slot):
        pltpu.make_async_copy(node_hbm.at[sidx_v.at[slot]], rows_v.at[slot], gsem.at[slot]).wait()

    load_idx_async(0, 0)
    wait_idx(0, 0)
    fire_gather(0)
    load_idx_async(1, 1)

    def ebody(j, carry):
        def iteration(p, pn):
            drain_gather(p)

            @pl.when(j + 1 < _NCHUNK)
            def _next_gather():
                wait_idx(j + 1, pn)
                fire_gather(pn)

            pltpu.sync_copy(rows_v.at[p], acc_sh.at[didx_v.at[p]], add=True)

            @pl.when(j + 2 < _NCHUNK)
            def _prefetch_idx():
                load_idx_async(j + 2, p)

        @pl.when(lax.rem(j, 2) == 0)
        def _even():
            iteration(0, 1)

        @pl.when(lax.rem(j, 2) == 1)
        def _odd():
            iteration(1, 0)

        return carry

    lax.fori_loop(0, _NCHUNK, ebody, 0)
    plsc.subcore_barrier()

    def obody(k, carry):
        q = k * _NS + sid

        @pl.when(q < _NWC)
        def _oc():
            pltpu.sync_copy(
                acc_sh.at[pl.ds(q * _WC, _WC)],
                out_hbm.at[cid].at[pl.ds(q * _WC, _WC)],
            )

        return carry

    lax.fori_loop(0, _NWK, obody, 0)


def _sc_agg(node, src, dst):
    mesh = plsc.VectorSubcoreMesh(
        core_axis_name="c", subcore_axis_name="s",
        num_cores=_NC, num_subcores=_NS,
    )
    run = pl.kernel(
        _sc_agg_body,
        out_type=jax.ShapeDtypeStruct((_NC, _N, _D), jnp.float32),
        mesh=mesh,
        scratch_types=[
            pltpu.VMEM((2, _CH), jnp.int32),
            pltpu.VMEM((2, _CH), jnp.int32),
            pltpu.VMEM((2, _CH, _D), jnp.float32),
            pltpu.VMEM((_WC, _D), jnp.float32),
            pltpu.VMEM_SHARED((_N, _D), jnp.float32),
            pltpu.SemaphoreType.DMA((2,)),
            pltpu.SemaphoreType.DMA((2,)),
        ],
    )
    return run(node, src, dst)


_RT = 400
_NT = _N // _RT


def _gnn_body(node_r, p0_r, p1_r, w1_r, w2_r, b_r, x_r):
    agg = p0_r[...] + p1_r[...]
    acc = jnp.dot(node_r[...], w1_r[...], preferred_element_type=jnp.float32)
    acc = acc + jnp.dot(agg, w2_r[...], preferred_element_type=jnp.float32)
    x_r[...] = jnp.maximum(acc + b_r[...], 0.0)


def _gnn(node, p0, p1, w1, w2, b2d):
    return pl.pallas_call(
        _gnn_body,
        grid=(_NT,),
        in_specs=[
            pl.BlockSpec((_RT, _D), lambda i: (i, 0)),
            pl.BlockSpec((_RT, _D), lambda i: (i, 0)),
            pl.BlockSpec((_RT, _D), lambda i: (i, 0)),
            pl.BlockSpec((_D, _D), lambda i: (0, 0)),
            pl.BlockSpec((_D, _D), lambda i: (0, 0)),
            pl.BlockSpec((1, _D), lambda i: (0, 0)),
        ],
        out_specs=pl.BlockSpec((_RT, _D), lambda i: (i, 0)),
        out_shape=jax.ShapeDtypeStruct((_N, _D), jnp.float32),
    )(node, p0, p1, w1, w2, b2d)


def _lstm_attn_body(
    starts_s, counts_s, tmax_s,
    x_r, wi_r, wh_r, bl_r, wa_r, va_r,
    out_r, aw_r,
    h_r, c_r, xt_r, xu_r, nl_r, sc_r,
):
    h_r[...] = jnp.zeros((_B, _H), jnp.float32)
    c_r[...] = jnp.zeros((_B, _H), jnp.float32)

    def gather_rows(t, buf):
        for b in range(_B):
            idx = jnp.minimum(starts_s[b] + t, _N - 1)
            buf[pl.ds(b, 1), :] = x_r[pl.ds(idx, 1), :]

    def halfstep(t, cur, nxt):
        gates = (
            jnp.dot(cur[...], wi_r[...], preferred_element_type=jnp.float32)
            + jnp.dot(h_r[...], wh_r[...], preferred_element_type=jnp.float32)
            + bl_r[...]
        )
        gather_rows(t + 1, nxt)
        i_g = jax.nn.sigmoid(gates[:, 0:_H])
        f_g = jax.nn.sigmoid(gates[:, _H:2 * _H])
        g_g = jnp.tanh(gates[:, 2 * _H:3 * _H])
        o_g = jax.nn.sigmoid(gates[:, 3 * _H:4 * _H])
        c_new = f_g * c_r[...] + i_g * g_g
        h_new = o_g * jnp.tanh(c_new)
        h_r[...] = h_new
        c_r[...] = c_new
        for b in range(_B):
            dst_row = jnp.where(t < counts_s[b], starts_s[b] + t, _NP - 1)
            nl_r[pl.ds(dst_row, 1), :] = h_new[b:b + 1, :]

    gather_rows(0, xt_r)

    def step(t, carry):
        @pl.when(lax.rem(t, 2) == 0)
        def _even():
            halfstep(t, xt_r, xu_r)

        @pl.when(lax.rem(t, 2) == 1)
        def _odd():
            halfstep(t, xu_r, xt_r)

        return carry

    lax.fori_loop(0, tmax_s[0], step, 0)

    def score_tile(j, carry):
        rows = pl.ds(j * _RT, _RT)
        tt = jnp.tanh(jnp.dot(nl_r[rows, :], wa_r[...], preferred_element_type=jnp.float32))
        sc_r[rows, :] = jnp.sum(tt * va_r[...], axis=1, keepdims=True)
        return carry

    lax.fori_loop(0, _NT, score_tile, 0)

    iota = lax.broadcasted_iota(jnp.int32, (128, 1), 0)
    neg_inf11 = jnp.full((1, 1), -jnp.inf, jnp.float32)
    neg_inf1h = jnp.full((1, _H), -jnp.inf, jnp.float32)

    def graph_body(b, carry):
        s0 = starts_s[b]
        cnt = counts_s[b]

        @pl.when(cnt > 0)
        def _nonempty():
            ktiles = (cnt + 127) // 128

            def max_tile(k, m):
                tile = sc_r[pl.ds(s0 + k * 128, 128), :]
                msk = iota < (cnt - k * 128)
                return jnp.maximum(m, jnp.max(jnp.where(msk, tile, -jnp.inf), axis=0, keepdims=True))

            m = lax.fori_loop(0, ktiles, max_tile, neg_inf11)

            def sum_tile(k, acc):
                tile = sc_r[pl.ds(s0 + k * 128, 128), :]
                msk = iota < (cnt - k * 128)
                return acc + jnp.sum(jnp.where(msk, jnp.exp(tile - m), 0.0), axis=0, keepdims=True)

            den = lax.fori_loop(0, ktiles, sum_tile, jnp.zeros((1, 1), jnp.float32))
            inv = 1.0 / den

            def pool_tile(k, acc):
                rows = pl.ds(s0 + k * 128, 128)
                w = jnp.exp(sc_r[rows, :] - m) * inv
                aw_r[rows, :] = w
                na = nl_r[rows, :] * w
                msk = iota < (cnt - k * 128)
                na = jnp.where(msk, na, -jnp.inf)
                return jnp.maximum(acc, jnp.max(na, axis=0, keepdims=True))

            pooled = lax.fori_loop(0, ktiles, pool_tile, neg_inf1h)
            out_r[pl.ds(b, 1), :] = pooled

        @pl.when(cnt == 0)
        def _empty():
            out_r[pl.ds(b, 1), :] = jnp.zeros((1, _H), jnp.float32)

        return carry

    lax.fori_loop(0, _B, graph_body, 0)


def _lstm_attn(starts, counts, tmax, x, wi, wh, bl2d, wa, va2d):
    return pl.pallas_call(
        _lstm_attn_body,
        in_specs=[
            pl.BlockSpec(memory_space=pltpu.SMEM),
            pl.BlockSpec(memory_space=pltpu.SMEM),
            pl.BlockSpec(memory_space=pltpu.SMEM),
            pl.BlockSpec(memory_space=pltpu.VMEM),
            pl.BlockSpec(memory_space=pltpu.VMEM),
            pl.BlockSpec(memory_space=pltpu.VMEM),
            pl.BlockSpec(memory_space=pltpu.VMEM),
            pl.BlockSpec(memory_space=pltpu.VMEM),
            pl.BlockSpec(memory_space=pltpu.VMEM),
        ],
        out_specs=[
            pl.BlockSpec(memory_space=pltpu.VMEM),
            pl.BlockSpec(memory_space=pltpu.VMEM),
        ],
        out_shape=[
            jax.ShapeDtypeStruct((_B, _H), jnp.float32),
            jax.ShapeDtypeStruct((_NP, 1), jnp.float32),
        ],
        scratch_shapes=[
            pltpu.VMEM((_B, _H), jnp.float32),
            pltpu.VMEM((_B, _H), jnp.float32),
            pltpu.VMEM((_B, _D), jnp.float32),
            pltpu.VMEM((_B, _D), jnp.float32),
            pltpu.VMEM((_NP, _H), jnp.float32),
            pltpu.VMEM((_NP, 1), jnp.float32),
        ],
    )(starts, counts, tmax, x, wi, wh, bl2d, wa, va2d)


def kernel(node, edge_index, batch, W1, W2, b_gnn, Wi, Wh, b_lstm, Wa, va):
    src = edge_index[0]
    dst = edge_index[1]

    ids = jnp.arange(_B, dtype=batch.dtype)
    starts = jnp.searchsorted(batch, ids, side="left").astype(jnp.int32)
    ends = jnp.searchsorted(batch, ids, side="right").astype(jnp.int32)
    counts = ends - starts
    tmax = jnp.max(counts).reshape(1)

    parts = _sc_agg(node, src, dst)
    x = _gnn(node, parts[0], parts[1], W1, W2, b_gnn.reshape(1, _D))
    out, aw = _lstm_attn(
        starts, counts, tmax, x,
        Wi, Wh, b_lstm.reshape(1, 4 * _H),
        Wa, va.reshape(1, _H),
    )
    return out, aw[:_N, 0]


# speedup vs baseline: 1.0608x; 1.0579x over previous
import functools

import jax
import jax.numpy as jnp
from jax import lax
from jax.experimental import pallas as pl
from jax.experimental.pallas import tpu as pltpu
from jax.experimental.pallas import tpu_sc as plsc

_N = 10000
_E = 320000
_D = 128
_H = 128
_B = 64

_NP = 10240

_NC = 2
_NS = 16
_NW = _NC * _NS
_EW = _E // _NW
_CH = 80
_NCHUNK = _EW // _CH
_WC = 80
_NWC = _N // _WC
_NWK = -(-_NWC // _NS)

def _sc_agg_body(node_hbm, src_hbm, dst_hbm, out_hbm, sidx_v, didx_v, rows_v, zero_v, acc_sh, isem, gsem):
    cid = lax.axis_index("c")
    sid = lax.axis_index("s")

    def zrow(r, carry):
        for j in range(_D // 16):
            zero_v[r, pl.ds(j * 16, 16)] = jnp.zeros((16,), jnp.float32)
        return carry

    lax.fori_loop(0, _WC, zrow, 0)

    def zbody(k, carry):
        q = k * _NS + sid

        @pl.when(q < _NWC)
        def _zc():
            pltpu.sync_copy(zero_v, acc_sh.at[pl.ds(q * _WC, _WC)])

        return carry

    lax.fori_loop(0, _NWK, zbody, 0)
    plsc.subcore_barrier()

    wid = sid * _NC + cid
    ebase = wid * _EW

    def load_idx_async(j, slot):
        off = ebase + j * _CH
        pltpu.async_copy(src_hbm.at[pl.ds(off, _CH)], sidx_v.at[slot], isem.at[slot])
        pltpu.async_copy(dst_hbm.at[pl.ds(off, _CH)], didx_v.at[slot], isem.at[slot])

    def wait_idx(j, slot):
        off = ebase + j * _CH
        pltpu.make_async_copy(src_hbm.at[pl.ds(off, _CH)], sidx_v.at[slot], isem.at[slot]).wait()
        pltpu.make_async_copy(dst_hbm.at[pl.ds(off, _CH)], didx_v.at[slot], isem.at[slot]).wait()

    def fire_gather(slot):
        pltpu.async_copy(node_hbm.at[sidx_v.at[slot]], rows_v.at[slot], gsem.at[slot])

    def drain_gather(slot):
        pltpu.make_async_copy(node_hbm.at[sidx_v.at[slot]], rows_v.at[slot], gsem.at[slot]).wait()

    load_idx_async(0, 0)
    wait_idx(0, 0)
    fire_gather(0)
    load_idx_async(1, 1)

    def ebody(j, carry):
        def iteration(p, pn):
            drain_gather(p)

            @pl.when(j + 1 < _NCHUNK)
            def _next_gather():
                wait_idx(j + 1, pn)
                fire_gather(pn)

            pltpu.sync_copy(rows_v.at[p], acc_sh.at[didx_v.at[p]], add=True)

            @pl.when(j + 2 < _NCHUNK)
            def _prefetch_idx():
                load_idx_async(j + 2, p)

        @pl.when(lax.rem(j, 2) == 0)
        def _even():
            iteration(0, 1)

        @pl.when(lax.rem(j, 2) == 1)
        def _odd():
            iteration(1, 0)

        return carry

    lax.fori_loop(0, _NCHUNK, ebody, 0)
    plsc.subcore_barrier()

    def obody(k, carry):
        q = k * _NS + sid

        @pl.when(q < _NWC)
        def _oc():
            pltpu.sync_copy(
                acc_sh.at[pl.ds(q * _WC, _WC)],
                out_hbm.at[cid].at[pl.ds(q * _WC, _WC)],
            )

        return carry

    lax.fori_loop(0, _NWK, obody, 0)


def _sc_agg(node, src, dst):
    mesh = plsc.VectorSubcoreMesh(
        core_axis_name="c", subcore_axis_name="s",
        num_cores=_NC, num_subcores=_NS,
    )
    run = pl.kernel(
        _sc_agg_body,
        out_type=jax.ShapeDtypeStruct((_NC, _N, _D), jnp.float32),
        mesh=mesh,
        scratch_types=[
            pltpu.VMEM((2, _CH), jnp.int32),
            pltpu.VMEM((2, _CH), jnp.int32),
            pltpu.VMEM((2, _CH, _D), jnp.float32),
            pltpu.VMEM((_WC, _D), jnp.float32),
            pltpu.VMEM_SHARED((_N, _D), jnp.float32),
            pltpu.SemaphoreType.DMA((2,)),
            pltpu.SemaphoreType.DMA((2,)),
        ],
    )
    return run(node, src, dst)


_RT = 400
_NT = _N // _RT


def _gnn_body(node_r, p0_r, p1_r, w1_r, w2_r, b_r, x_r):
    agg = p0_r[...] + p1_r[...]
    acc = jnp.dot(node_r[...], w1_r[...], preferred_element_type=jnp.float32)
    acc = acc + jnp.dot(agg, w2_r[...], preferred_element_type=jnp.float32)
    x_r[...] = jnp.maximum(acc + b_r[...], 0.0)


def _gnn(node, p0, p1, w1, w2, b2d):
    return pl.pallas_call(
        _gnn_body,
        grid=(_NT,),
        in_specs=[
            pl.BlockSpec((_RT, _D), lambda i: (i, 0)),
            pl.BlockSpec((_RT, _D), lambda i: (i, 0)),
            pl.BlockSpec((_RT, _D), lambda i: (i, 0)),
            pl.BlockSpec((_D, _D), lambda i: (0, 0)),
            pl.BlockSpec((_D, _D), lambda i: (0, 0)),
            pl.BlockSpec((1, _D), lambda i: (0, 0)),
        ],
        out_specs=pl.BlockSpec((_RT, _D), lambda i: (i, 0)),
        out_shape=jax.ShapeDtypeStruct((_N, _D), jnp.float32),
    )(node, p0, p1, w1, w2, b2d)


def _lstm_attn_body(
    starts_s, counts_s, tmax_s,
    x_r, wi_r, wh_r, bl_r, wa_r, va_r,
    out_r, aw_r,
    h_r, c_r, xt_r, nl_r, sc_r,
):
    h_r[...] = jnp.zeros((_B, _H), jnp.float32)
    c_r[...] = jnp.zeros((_B, _H), jnp.float32)

    def step(t, carry):
        for b in range(_B):
            idx = jnp.minimum(starts_s[b] + t, _N - 1)
            xt_r[pl.ds(b, 1), :] = x_r[pl.ds(idx, 1), :]
        gates = (
            jnp.dot(xt_r[...], wi_r[...], preferred_element_type=jnp.float32)
            + jnp.dot(h_r[...], wh_r[...], preferred_element_type=jnp.float32)
            + bl_r[...]
        )
        i_g = jax.nn.sigmoid(gates[:, 0:_H])
        f_g = jax.nn.sigmoid(gates[:, _H:2 * _H])
        g_g = jnp.tanh(gates[:, 2 * _H:3 * _H])
        o_g = jax.nn.sigmoid(gates[:, 3 * _H:4 * _H])
        c_new = f_g * c_r[...] + i_g * g_g
        h_new = o_g * jnp.tanh(c_new)
        h_r[...] = h_new
        c_r[...] = c_new
        for b in range(_B):
            dst_row = jnp.where(t < counts_s[b], starts_s[b] + t, _NP - 1)
            nl_r[pl.ds(dst_row, 1), :] = h_new[b:b + 1, :]
        return carry

    lax.fori_loop(0, tmax_s[0], step, 0)

    def score_tile(j, carry):
        rows = pl.ds(j * _RT, _RT)
        tt = jnp.tanh(jnp.dot(nl_r[rows, :], wa_r[...], preferred_element_type=jnp.float32))
        sc_r[rows, :] = jnp.sum(tt * va_r[...], axis=1, keepdims=True)
        return carry

    lax.fori_loop(0, _NT, score_tile, 0)

    iota = lax.broadcasted_iota(jnp.int32, (128, 1), 0)
    neg_inf11 = jnp.full((1, 1), -jnp.inf, jnp.float32)
    neg_inf1h = jnp.full((1, _H), -jnp.inf, jnp.float32)

    def graph_body(b, carry):
        s0 = starts_s[b]
        cnt = counts_s[b]

        @pl.when(cnt > 0)
        def _nonempty():
            ktiles = (cnt + 127) // 128

            def max_tile(k, m):
                tile = sc_r[pl.ds(s0 + k * 128, 128), :]
                msk = iota < (cnt - k * 128)
                return jnp.maximum(m, jnp.max(jnp.where(msk, tile, -jnp.inf), axis=0, keepdims=True))

            m = lax.fori_loop(0, ktiles, max_tile, neg_inf11)

            def sum_tile(k, acc):
                tile = sc_r[pl.ds(s0 + k * 128, 128), :]
                msk = iota < (cnt - k * 128)
                return acc + jnp.sum(jnp.where(msk, jnp.exp(tile - m), 0.0), axis=0, keepdims=True)

            den = lax.fori_loop(0, ktiles, sum_tile, jnp.zeros((1, 1), jnp.float32))
            inv = 1.0 / den

            def pool_tile(k, acc):
                rows = pl.ds(s0 + k * 128, 128)
                w = jnp.exp(sc_r[rows, :] - m) * inv
                aw_r[rows, :] = w
                na = nl_r[rows, :] * w
                msk = iota < (cnt - k * 128)
                na = jnp.where(msk, na, -jnp.inf)
                return jnp.maximum(acc, jnp.max(na, axis=0, keepdims=True))

            pooled = lax.fori_loop(0, ktiles, pool_tile, neg_inf1h)
            out_r[pl.ds(b, 1), :] = pooled

        @pl.when(cnt == 0)
        def _empty():
            out_r[pl.ds(b, 1), :] = jnp.zeros((1, _H), jnp.float32)

        return carry

    lax.fori_loop(0, _B, graph_body, 0)


def _lstm_attn(starts, counts, tmax, x, wi, wh, bl2d, wa, va2d):
    return pl.pallas_call(
        _lstm_attn_body,
        in_specs=[
            pl.BlockSpec(memory_space=pltpu.SMEM),
            pl.BlockSpec(memory_space=pltpu.SMEM),
            pl.BlockSpec(memory_space=pltpu.SMEM),
            pl.BlockSpec(memory_space=pltpu.VMEM),
            pl.BlockSpec(memory_space=pltpu.VMEM),
            pl.BlockSpec(memory_space=pltpu.VMEM),
            pl.BlockSpec(memory_space=pltpu.VMEM),
            pl.BlockSpec(memory_space=pltpu.VMEM),
            pl.BlockSpec(memory_space=pltpu.VMEM),
        ],
        out_specs=[
            pl.BlockSpec(memory_space=pltpu.VMEM),
            pl.BlockSpec(memory_space=pltpu.VMEM),
        ],
        out_shape=[
            jax.ShapeDtypeStruct((_B, _H), jnp.float32),
            jax.ShapeDtypeStruct((_NP, 1), jnp.float32),
        ],
        scratch_shapes=[
            pltpu.VMEM((_B, _H), jnp.float32),
            pltpu.VMEM((_B, _H), jnp.float32),
            pltpu.VMEM((_B, _D), jnp.float32),
            pltpu.VMEM((_NP, _H), jnp.float32),
            pltpu.VMEM((_NP, 1), jnp.float32),
        ],
    )(starts, counts, tmax, x, wi, wh, bl2d, wa, va2d)


def kernel(node, edge_index, batch, W1, W2, b_gnn, Wi, Wh, b_lstm, Wa, va):
    src = edge_index[0]
    dst = edge_index[1]

    ids = jnp.arange(_B, dtype=batch.dtype)
    starts = jnp.searchsorted(batch, ids, side="left").astype(jnp.int32)
    ends = jnp.searchsorted(batch, ids, side="right").astype(jnp.int32)
    counts = ends - starts
    tmax = jnp.max(counts).reshape(1)

    parts = _sc_agg(node, src, dst)
    x = _gnn(node, parts[0], parts[1], W1, W2, b_gnn.reshape(1, _D))
    out, aw = _lstm_attn(
        starts, counts, tmax, x,
        Wi, Wh, b_lstm.reshape(1, 4 * _H),
        Wa, va.reshape(1, _H),
    )
    return out, aw[:_N, 0]
